# Initial kernel scaffold; baseline (speedup 1.0000x reference)
import jax
import jax.numpy as jnp
from jax.experimental import pallas as pl


def kernel(edge_vec, pos, q, w_edge, kvecs, cell_volume, edge_index, batch, num_atoms):
    raise NotImplementedError("write your pallas kernel here")


# speedup vs baseline: 2.0364x; 2.0364x over previous
import jax
import jax.numpy as jnp
from jax import lax
from jax.experimental import pallas as pl
from jax.experimental.pallas import tpu as pltpu
from jax.experimental.pallas import tpu_sc as plsc

N = 50000
E = 1600000
NB = 64
K = 8

NC = 2
NS = 16
NW = NC * NS
CB = 128
SJ = 10
C = CB * SJ
NCHUNK = E // C
TMAX = (NCHUNK + NW - 1) // NW
NPAD = 51200
APT = 3 * NPAD // NS
VACC = NB * 6 * 16


def _sc_body(ev_hbm, srcf_hbm, dstf_hbm, batch_hbm, wb_hbm,
             fpart_hbm, spart_hbm,
             accum_sh, batch_v, ev_v, srcflat_v, dstflat_v,
             isrc_v, idst_v, vsrc_v, vdst_v, vacc_v, wb_v, bounce_v, sem):
    core = lax.axis_index("c")
    sid = lax.axis_index("s")
    wid = core * NS + sid

    lane = lax.iota(jnp.int32, 16)
    lane3 = lane * 3
    lane384 = lane * 384
    zf = jnp.zeros((16,), jnp.float32)

    pltpu.sync_copy(batch_hbm, batch_v)
    pltpu.sync_copy(wb_hbm, wb_v)

    def zero_bounce(i, carry):
        bounce_v[pl.ds(i * 16, 16)] = zf
        return carry
    lax.fori_loop(0, APT // 16, zero_bounce, 0)
    pltpu.sync_copy(bounce_v, accum_sh.at[pl.ds(sid * APT, APT)])

    def zero_vacc(i, carry):
        vacc_v[pl.ds(i * 16, 16)] = zf
        return carry
    lax.fori_loop(0, VACC // 16, zero_vacc, 0)

    plsc.subcore_barrier()

    wx = wb_v[0, :]
    wy = wb_v[1, :]
    wz = wb_v[2, :]

    def chunk_body(t, carry):
        cid = t * NW + wid

        @pl.when(cid < NCHUNK)
        def _():
            pltpu.sync_copy(ev_hbm.at[pl.ds(cid * (3 * C), 3 * C)], ev_v)
            pltpu.sync_copy(srcf_hbm.at[pl.ds(cid * C, C)], srcflat_v)
            pltpu.sync_copy(dstf_hbm.at[pl.ds(cid * C, C)], dstflat_v)

            def subchunk(j, jcarry):
                for g in range(CB // 16):
                    el = j * CB + g * 16
                    r0 = g * 16
                    s16 = srcflat_v[pl.ds(el, 16)]
                    d16 = dstflat_v[pl.ds(el, 16)]
                    ix = el * 3 + lane3
                    rx = plsc.load_gather(ev_v, [ix])
                    ry = plsc.load_gather(ev_v, [ix + 1])
                    rz = plsc.load_gather(ev_v, [ix + 2])
                    r2 = rx * rx + ry * ry + rz * rz
                    a = jnp.exp(r2 * -0.1) * -0.2
                    tt = rx * wx + ry * wy + rz * wz
                    eb = jnp.exp(jnp.abs(tt) * -2.0)
                    dd = eb + 1.0
                    s = (4.0 * eb) / (dd * dd)
                    fx = a * rx + s * wx
                    fy = a * ry + s * wy
                    fz = a * rz + s * wz

                    isrc_v[0, pl.ds(r0, 16)] = s16
                    isrc_v[1, pl.ds(r0, 16)] = s16 + NPAD
                    isrc_v[2, pl.ds(r0, 16)] = s16 + 2 * NPAD
                    idst_v[0, pl.ds(r0, 16)] = d16
                    idst_v[1, pl.ds(r0, 16)] = d16 + NPAD
                    idst_v[2, pl.ds(r0, 16)] = d16 + 2 * NPAD
                    vsrc_v[0, pl.ds(r0, 16)] = fx
                    vsrc_v[1, pl.ds(r0, 16)] = fy
                    vsrc_v[2, pl.ds(r0, 16)] = fz
                    vdst_v[0, pl.ds(r0, 16)] = -fx
                    vdst_v[1, pl.ds(r0, 16)] = -fy
                    vdst_v[2, pl.ds(r0, 16)] = -fz

                    seg = plsc.load_gather(batch_v, [d16])
                    sb = lane384 + seg * 6
                    plsc.addupdate_scatter(vacc_v, [sb], rx * fx)
                    plsc.addupdate_scatter(vacc_v, [sb + 1], ry * fy)
                    plsc.addupdate_scatter(vacc_v, [sb + 2], rz * fz)
                    plsc.addupdate_scatter(vacc_v, [sb + 3], rx * fy)
                    plsc.addupdate_scatter(vacc_v, [sb + 4], ry * fz)
                    plsc.addupdate_scatter(vacc_v, [sb + 5], rz * fx)

                copies = []
                for c in range(3):
                    copies.append(pltpu.async_copy(
                        vsrc_v.at[c], accum_sh.at[isrc_v.at[c]], sem,
                        add=True))
                    copies.append(pltpu.async_copy(
                        vdst_v.at[c], accum_sh.at[idst_v.at[c]], sem,
                        add=True))
                for cp in copies:
                    cp.wait()
                return jcarry

            lax.fori_loop(0, SJ, subchunk, 0)
        return carry

    lax.fori_loop(0, TMAX, chunk_body, 0)

    plsc.subcore_barrier()

    pltpu.sync_copy(accum_sh.at[pl.ds(sid * APT, APT)], bounce_v)
    pltpu.sync_copy(bounce_v, fpart_hbm.at[core, pl.ds(sid * APT, APT)])
    pltpu.sync_copy(vacc_v, spart_hbm.at[wid])


def _sc_scatter(ev_flat, srcflat, dstflat, batch, wb):
    mesh = plsc.VectorSubcoreMesh(core_axis_name="c", subcore_axis_name="s")
    f = pl.kernel(
        _sc_body,
        out_type=(
            jax.ShapeDtypeStruct((NC, 3 * NPAD), jnp.float32),
            jax.ShapeDtypeStruct((NW, VACC), jnp.float32),
        ),
        mesh=mesh,
        compiler_params=pltpu.CompilerParams(needs_layout_passes=False),
        scratch_types=[
            pltpu.VMEM_SHARED((3 * NPAD,), jnp.float32),
            pltpu.VMEM((N,), jnp.int32),
            pltpu.VMEM((3 * C,), jnp.float32),
            pltpu.VMEM((C,), jnp.int32),
            pltpu.VMEM((C,), jnp.int32),
            pltpu.VMEM((3, CB), jnp.int32),
            pltpu.VMEM((3, CB), jnp.int32),
            pltpu.VMEM((3, CB), jnp.float32),
            pltpu.VMEM((3, CB), jnp.float32),
            pltpu.VMEM((VACC,), jnp.float32),
            pltpu.VMEM((3, 16), jnp.float32),
            pltpu.VMEM((APT,), jnp.float32),
            pltpu.SemaphoreType.DMA,
        ],
    )
    return f(ev_flat, srcflat, dstflat, batch, wb)


def _moments_body(kv_ref, pos_ref, q_ref, out_ref):
    ph = jnp.dot(kv_ref[...], pos_ref[...],
                 preferred_element_type=jnp.float32)
    qv = q_ref[...]
    out_ref[:, 0:1] = jnp.sum(qv * jnp.cos(ph), axis=1, keepdims=True)
    out_ref[:, 1:2] = jnp.sum(qv * jnp.sin(ph), axis=1, keepdims=True)


def _finalize_body(fp0_ref, fp1_ref, pos_ref, q_ref, kv_ref, kvt_ref,
                   srsi_ref, force_ref):
    ph = jnp.dot(kv_ref[...], pos_ref[...],
                 preferred_element_type=jnp.float32)
    cre = srsi_ref[:, 0:1]
    cim = srsi_ref[:, 1:2]
    gmat = 2.0 * (cim * jnp.cos(ph) - cre * jnp.sin(ph))
    pg = jnp.dot(kvt_ref[...], gmat,
                 preferred_element_type=jnp.float32) * q_ref[...]
    force_ref[...] = (fp0_ref[...] + fp1_ref[...]) - pg


def _stress_body(sp_ref, vol_ref, out_ref):
    out_ref[...] = -(jnp.sum(sp_ref[...], axis=0, keepdims=True)
                     / vol_ref[...])


def kernel(edge_vec, pos, q, w_edge, kvecs, cell_volume, edge_index, batch,
           num_atoms):
    ev_flat = edge_vec.reshape(-1)
    srcflat = edge_index[0]
    dstflat = edge_index[1]
    wb = jnp.broadcast_to(w_edge.reshape(3, 1), (3, 16))

    fpart, spart = _sc_scatter(ev_flat, srcflat, dstflat, batch, wb)

    pos_t = pos.T
    q_t = q.reshape(1, N)
    kvt = kvecs.T
    srsi = pl.pallas_call(
        _moments_body,
        out_shape=jax.ShapeDtypeStruct((K, 2), jnp.float32),
    )(kvecs, pos_t, q_t)

    fp = fpart.reshape(NC, 3, NPAD)
    fp0 = fp[0]
    fp1 = fp[1]
    pos_tp = jnp.pad(pos_t, ((0, 0), (0, NPAD - N)))
    q_tp = jnp.pad(q_t, ((0, 0), (0, NPAD - N)))

    RC = 2048
    grid = NPAD // RC
    force_t = pl.pallas_call(
        _finalize_body,
        grid=(grid,),
        in_specs=[
            pl.BlockSpec((3, RC), lambda i: (0, i)),
            pl.BlockSpec((3, RC), lambda i: (0, i)),
            pl.BlockSpec((3, RC), lambda i: (0, i)),
            pl.BlockSpec((1, RC), lambda i: (0, i)),
            pl.BlockSpec((K, 3), lambda i: (0, 0)),
            pl.BlockSpec((3, K), lambda i: (0, 0)),
            pl.BlockSpec((K, 2), lambda i: (0, 0)),
        ],
        out_specs=pl.BlockSpec((3, RC), lambda i: (0, i)),
        out_shape=jax.ShapeDtypeStruct((3, NPAD), jnp.float32),
    )(fp0, fp1, pos_tp, q_tp, kvecs, kvt, srsi)
    force = force_t[:, :N].T

    volrep = jnp.repeat(cell_volume, 6).reshape(1, NB * 6)
    stress_flat = pl.pallas_call(
        _stress_body,
        out_shape=jax.ShapeDtypeStruct((1, NB * 6), jnp.float32),
    )(spart.reshape(NW * 16, NB * 6), volrep)
    stress = stress_flat.reshape(NB, 6)

    return force, stress
